# baseline (speedup 1.0000x reference)
import numpy as np
import jax
import jax.numpy as jnp
from jax.experimental import pallas as pl
from jax.experimental.pallas import tpu as pltpu

_EPS = 1e-5
_T = 64
_ROWS = _T * 64

_S0 = np.zeros((64, 16, 9), np.float32)
for q in range(4):
    for ap in range(4):
        for bp in range(4):
            a = 4 * (q // 2) + ap
            b = 4 * (q % 2) + bp
            for ue in range(4):
                for vf in range(4):
                    i, j = a - ue, b - vf
                    if 0 <= i < 3 and 0 <= j < 3:
                        _S0[q * 16 + ap * 4 + bp, ue * 4 + vf, 3 * i + j] = 1.0
_S1 = np.zeros((16, 4, 2, 2), np.float32)
for e in range(2):
    for f in range(2):
        for dr in range(2):
            for dc in range(2):
                _S1[(2 * e + dr) * 4 + (2 * f + dc), e * 2 + f, dr, dc] = 1.0
_CMASK = np.zeros((64, 1), np.float32)
for R in range(7):
    for C in range(7):
        _CMASK[R * 8 + C, 0] = 1.0
_MASK = np.tile(_CMASK, (_T, 1))
_APOOL = np.kron(np.eye(_T, dtype=np.float32),
                 (_CMASK.T / 49.0).astype(np.float32))


def _k_stats0(x_ref, w_ref, m_ref, s_ref, q_ref):
    h = jnp.dot(x_ref[...], w_ref[...], preferred_element_type=jnp.float32)
    hm = h * m_ref[...]
    s_ref[0, 0, :] = jnp.sum(hm, axis=0)
    q_ref[0, 0, :] = jnp.sum(hm * h, axis=0)


def _k_stage1(x_ref, w0_ref, sh0_ref, w1a_ref, w1b_ref, m_ref,
              h1_ref, s_ref, q_ref):
    h0 = jnp.maximum(
        jnp.dot(x_ref[...], w0_ref[...], preferred_element_type=jnp.float32)
        + sh0_ref[0], 0.0)
    h1a = jnp.dot(h0[:, 0:256], w1a_ref[...],
                  preferred_element_type=jnp.float32)
    h1b = jnp.dot(h0[:, 256:512], w1b_ref[...],
                  preferred_element_type=jnp.float32)
    h1 = jnp.concatenate([h1a, h1b], axis=1)
    h1_ref[...] = h1
    hm = h1 * m_ref[...]
    s_ref[0, 0, :] = jnp.sum(hm, axis=0)
    q_ref[0, 0, :] = jnp.sum(hm * h1, axis=0)


def _k_stage2(h1_ref, sc1_ref, sh1_ref, w2_ref, m_ref, h2_ref, s_ref, q_ref):
    h1 = jnp.maximum(h1_ref[...] * sc1_ref[0] + sh1_ref[0], 0.0)
    h2 = jnp.dot(h1, w2_ref[...], preferred_element_type=jnp.float32)
    h2_ref[...] = h2
    hm = h2 * m_ref[...]
    s_ref[0, 0, :] = jnp.sum(hm, axis=0)
    q_ref[0, 0, :] = jnp.sum(hm * h2, axis=0)


def _k_stage3(h2_ref, sc2_ref, sh2_ref, ap_ref, wfc_ref, bfc_ref, o_ref):
    h2 = jnp.maximum(h2_ref[...] * sc2_ref[0] + sh2_ref[0], 0.0)
    z = jnp.dot(h2, wfc_ref[...], preferred_element_type=jnp.float32)
    o_ref[...] = jnp.dot(ap_ref[...], z,
                         preferred_element_type=jnp.float32) + bfc_ref[0]


def _scale_shift(s, q, n, g, be):
    mean = s / n
    var = q / n - mean * mean
    scale = g * jax.lax.rsqrt(var + _EPS)
    return scale, be - mean * scale


def kernel(x, W0, b0, g0, be0, W1, b1, g1, be1, W2, b2, g2, be2, Wfc, bfc):
    B = x.shape[0]
    nT = B // _T
    f32 = jnp.float32

    xpad = jnp.pad(x[:, 0], ((0, 0), (1, 7), (1, 7)))
    G = xpad.reshape(B, 9, 4, 9, 4).transpose(0, 1, 3, 2, 4)
    X = jnp.stack([G[:, dr:dr + 8, dc:dc + 8]
                   for dr in range(2) for dc in range(2)],
                  axis=3).reshape(B * 64, 64)

    w0r = jnp.transpose(W0[:, 0], (1, 2, 0)).reshape(9, 32)
    W0g = jnp.einsum('kpn,nc->kpc', jnp.asarray(_S0), w0r).reshape(64, 512)
    W1g = jnp.einsum('pqde,ocde->pcqo', jnp.asarray(_S1), W1).reshape(512, 256)
    W1a = W1g[0:256, 0:128]
    W1b = W1g[256:512, 128:256]
    W2g = jnp.transpose(W2, (2, 3, 1, 0)).reshape(256, 128)
    wfcT = jnp.transpose(Wfc)
    apool = jnp.asarray(_APOOL)
    mask = jnp.asarray(_MASK)

    cparams = pltpu.CompilerParams(dimension_semantics=("parallel",))

    s0, q0 = pl.pallas_call(
        _k_stats0,
        grid=(nT,),
        in_specs=[
            pl.BlockSpec((_ROWS, 64), lambda i: (i, 0)),
            pl.BlockSpec((64, 512), lambda i: (0, 0)),
            pl.BlockSpec((_ROWS, 1), lambda i: (0, 0)),
        ],
        out_specs=[
            pl.BlockSpec((1, 1, 512), lambda i: (i, 0, 0)),
            pl.BlockSpec((1, 1, 512), lambda i: (i, 0, 0)),
        ],
        out_shape=[
            jax.ShapeDtypeStruct((nT, 1, 512), f32),
            jax.ShapeDtypeStruct((nT, 1, 512), f32),
        ],
        compiler_params=cparams,
    )(X, W0g, mask)
    s0c = jnp.sum(s0, axis=(0, 1)).reshape(16, 32).sum(axis=0)
    q0c = jnp.sum(q0, axis=(0, 1)).reshape(16, 32).sum(axis=0)
    sc0, sh0 = _scale_shift(s0c, q0c, float(B * 28 * 28), g0, be0)
    W0s = W0g * jnp.tile(sc0, 16)[None, :]
    sh0t = jnp.tile(sh0, 16).reshape(1, 512)

    h1p, s1, q1 = pl.pallas_call(
        _k_stage1,
        grid=(nT,),
        in_specs=[
            pl.BlockSpec((_ROWS, 64), lambda i: (i, 0)),
            pl.BlockSpec((64, 512), lambda i: (0, 0)),
            pl.BlockSpec((1, 512), lambda i: (0, 0)),
            pl.BlockSpec((256, 128), lambda i: (0, 0)),
            pl.BlockSpec((256, 128), lambda i: (0, 0)),
            pl.BlockSpec((_ROWS, 1), lambda i: (0, 0)),
        ],
        out_specs=[
            pl.BlockSpec((_ROWS, 256), lambda i: (i, 0)),
            pl.BlockSpec((1, 1, 256), lambda i: (i, 0, 0)),
            pl.BlockSpec((1, 1, 256), lambda i: (i, 0, 0)),
        ],
        out_shape=[
            jax.ShapeDtypeStruct((B * 64, 256), f32),
            jax.ShapeDtypeStruct((nT, 1, 256), f32),
            jax.ShapeDtypeStruct((nT, 1, 256), f32),
        ],
        compiler_params=cparams,
    )(X, W0s, sh0t, W1a, W1b, mask)
    s1c = jnp.sum(s1, axis=(0, 1)).reshape(4, 64).sum(axis=0)
    q1c = jnp.sum(q1, axis=(0, 1)).reshape(4, 64).sum(axis=0)
    sc1, sh1 = _scale_shift(s1c, q1c, float(B * 14 * 14), g1, be1)
    sc1t = jnp.tile(sc1, 4).reshape(1, 256)
    sh1t = jnp.tile(sh1, 4).reshape(1, 256)

    h2p, s2, q2 = pl.pallas_call(
        _k_stage2,
        grid=(nT,),
        in_specs=[
            pl.BlockSpec((_ROWS, 256), lambda i: (i, 0)),
            pl.BlockSpec((1, 256), lambda i: (0, 0)),
            pl.BlockSpec((1, 256), lambda i: (0, 0)),
            pl.BlockSpec((256, 128), lambda i: (0, 0)),
            pl.BlockSpec((_ROWS, 1), lambda i: (0, 0)),
        ],
        out_specs=[
            pl.BlockSpec((_ROWS, 128), lambda i: (i, 0)),
            pl.BlockSpec((1, 1, 128), lambda i: (i, 0, 0)),
            pl.BlockSpec((1, 1, 128), lambda i: (i, 0, 0)),
        ],
        out_shape=[
            jax.ShapeDtypeStruct((B * 64, 128), f32),
            jax.ShapeDtypeStruct((nT, 1, 128), f32),
            jax.ShapeDtypeStruct((nT, 1, 128), f32),
        ],
        compiler_params=cparams,
    )(h1p, sc1t, sh1t, W2g, mask)
    sc2, sh2 = _scale_shift(jnp.sum(s2, axis=(0, 1)), jnp.sum(q2, axis=(0, 1)),
                            float(B * 7 * 7), g2, be2)

    out = pl.pallas_call(
        _k_stage3,
        grid=(nT,),
        in_specs=[
            pl.BlockSpec((_ROWS, 128), lambda i: (i, 0)),
            pl.BlockSpec((1, 128), lambda i: (0, 0)),
            pl.BlockSpec((1, 128), lambda i: (0, 0)),
            pl.BlockSpec((_T, _ROWS), lambda i: (0, 0)),
            pl.BlockSpec((128, 10), lambda i: (0, 0)),
            pl.BlockSpec((1, 10), lambda i: (0, 0)),
        ],
        out_specs=pl.BlockSpec((_T, 10), lambda i: (i, 0)),
        out_shape=jax.ShapeDtypeStruct((B, 10), f32),
        compiler_params=cparams,
    )(h2p, sc2.reshape(1, 128), sh2.reshape(1, 128), apool, wfcT,
      bfc.reshape(1, 10))
    return out


# speedup vs baseline: 3.8757x; 3.8757x over previous
import numpy as np
import jax
import jax.numpy as jnp
from jax.experimental import pallas as pl
from jax.experimental.pallas import tpu as pltpu

_EPS = 1e-5
_T = 64
_ROWS = _T * 8

_S0 = np.zeros((168, 7 * 16, 9), np.float32)
for C in range(7):
    for ue in range(4):
        for vf in range(4):
            for i in range(3):
                for j in range(3):
                    d = ue + i - 1
                    cx = 4 * C + vf + j - 1
                    if not (0 <= cx < 28):
                        continue
                    if d == -1:
                        k = cx
                    elif d <= 3:
                        k = 28 + 28 * d + cx
                    else:
                        k = 140 + cx
                    _S0[k, C * 16 + ue * 4 + vf, 3 * i + j] = 1.0
_S1 = np.zeros((16, 4, 2, 2), np.float32)
for e in range(2):
    for f in range(2):
        for dr in range(2):
            for dc in range(2):
                _S1[(2 * e + dr) * 4 + (2 * f + dc), e * 2 + f, dr, dc] = 1.0
_BMASK = np.tile(np.array([1.0] * 7 + [0.0], np.float32).reshape(8, 1),
                 (_T, 1))
_APOOL = np.kron(np.eye(_T, dtype=np.float32),
                 np.array([[1.0 / 49.0] * 7 + [0.0]], np.float32))


def _build_x(xs_ref):
    xb = xs_ref[...]
    A = xb[:, 0:8, 84:112]
    Bv = xb[:, 1:9, :]
    Cv = xb[:, 2:10, 0:28]
    return jnp.concatenate([A, Bv, Cv], axis=2).reshape(_ROWS, 168)


def _conv1(h0, w1a_ref, w1b_ref):
    parts = []
    for C in range(7):
        g = h0[:, C * 512:(C + 1) * 512]
        parts.append(jnp.dot(g[:, 0:256], w1a_ref[...],
                             preferred_element_type=jnp.float32))
        parts.append(jnp.dot(g[:, 256:512], w1b_ref[...],
                             preferred_element_type=jnp.float32))
    return jnp.concatenate(parts, axis=1)


def _conv2(h1, w2_ref):
    parts = []
    for C in range(7):
        g = h1[:, C * 256:(C + 1) * 256]
        parts.append(jnp.dot(g, w2_ref[...],
                             preferred_element_type=jnp.float32))
    return jnp.concatenate(parts, axis=1)


def _k_stats0(xs_ref, w_ref, m_ref, s_ref, q_ref):
    h = jnp.dot(_build_x(xs_ref), w_ref[...],
                preferred_element_type=jnp.float32)
    hm = h * m_ref[...]
    s_ref[0, 0, :] = jnp.sum(hm, axis=0)
    q_ref[0, 0, :] = jnp.sum(hm * h, axis=0)


def _k_stage1(xs_ref, w0_ref, sh0_ref, w1a_ref, w1b_ref, m_ref,
              h1_ref, s_ref, q_ref):
    h0 = jnp.maximum(
        jnp.dot(_build_x(xs_ref), w0_ref[...],
                preferred_element_type=jnp.float32) + sh0_ref[0], 0.0)
    h1 = _conv1(h0, w1a_ref, w1b_ref)
    h1_ref[...] = h1
    hm = h1 * m_ref[...]
    s_ref[0, 0, :] = jnp.sum(hm, axis=0)
    q_ref[0, 0, :] = jnp.sum(hm * h1, axis=0)


def _k_stage2(h1_ref, sc1_ref, sh1_ref, w2_ref, m_ref, h2_ref, s_ref, q_ref):
    h1 = jnp.maximum(h1_ref[...] * sc1_ref[0] + sh1_ref[0], 0.0)
    h2 = _conv2(h1, w2_ref)
    h2_ref[...] = h2
    hm = h2 * m_ref[...]
    s_ref[0, 0, :] = jnp.sum(hm, axis=0)
    q_ref[0, 0, :] = jnp.sum(hm * h2, axis=0)


def _k_stage3(h2_ref, sc2_ref, sh2_ref, ap_ref, wfc_ref, bfc_ref, o_ref):
    h2 = jnp.maximum(h2_ref[...] * sc2_ref[0] + sh2_ref[0], 0.0)
    acc = h2[:, 0:128]
    for C in range(1, 7):
        acc = acc + h2[:, C * 128:(C + 1) * 128]
    pooled = jnp.dot(ap_ref[...], acc, preferred_element_type=jnp.float32)
    o_ref[...] = jnp.dot(pooled, wfc_ref[...],
                         preferred_element_type=jnp.float32) + bfc_ref[0]


def _scale_shift(s, q, n, g, be):
    mean = s / n
    var = q / n - mean * mean
    scale = g * jax.lax.rsqrt(var + _EPS)
    return scale, be - mean * scale


def kernel(x, W0, b0, g0, be0, W1, b1, g1, be1, W2, b2, g2, be2, Wfc, bfc):
    B = x.shape[0]
    nT = B // _T
    f32 = jnp.float32

    xs = jnp.pad(x.reshape(B, 7, 112), ((0, 0), (1, 8), (0, 0)))

    w0r = jnp.transpose(W0[:, 0], (1, 2, 0)).reshape(9, 32)
    W0g = jnp.einsum('kpn,nc->kpc', jnp.asarray(_S0), w0r).reshape(168, 3584)
    W1g = jnp.einsum('pqde,ocde->pcqo', jnp.asarray(_S1), W1).reshape(512, 256)
    W1a = W1g[0:256, 0:128]
    W1b = W1g[256:512, 128:256]
    W2g = jnp.transpose(W2, (2, 3, 1, 0)).reshape(256, 128)
    wfcT = jnp.transpose(Wfc)
    apool = jnp.asarray(_APOOL)
    mask = jnp.asarray(_BMASK)

    cparams = pltpu.CompilerParams(dimension_semantics=("parallel",))

    s0, q0 = pl.pallas_call(
        _k_stats0,
        grid=(nT,),
        in_specs=[
            pl.BlockSpec((_T, 16, 112), lambda i: (i, 0, 0)),
            pl.BlockSpec((168, 3584), lambda i: (0, 0)),
            pl.BlockSpec((_ROWS, 1), lambda i: (0, 0)),
        ],
        out_specs=[
            pl.BlockSpec((1, 1, 3584), lambda i: (i, 0, 0)),
            pl.BlockSpec((1, 1, 3584), lambda i: (i, 0, 0)),
        ],
        out_shape=[
            jax.ShapeDtypeStruct((nT, 1, 3584), f32),
            jax.ShapeDtypeStruct((nT, 1, 3584), f32),
        ],
        compiler_params=cparams,
    )(xs, W0g, mask)
    s0c = jnp.sum(s0, axis=(0, 1)).reshape(112, 32).sum(axis=0)
    q0c = jnp.sum(q0, axis=(0, 1)).reshape(112, 32).sum(axis=0)
    sc0, sh0 = _scale_shift(s0c, q0c, float(B * 28 * 28), g0, be0)
    W0s = W0g * jnp.tile(sc0, 112)[None, :]
    sh0t = jnp.tile(sh0, 112).reshape(1, 3584)

    h1p, s1, q1 = pl.pallas_call(
        _k_stage1,
        grid=(nT,),
        in_specs=[
            pl.BlockSpec((_T, 16, 112), lambda i: (i, 0, 0)),
            pl.BlockSpec((168, 3584), lambda i: (0, 0)),
            pl.BlockSpec((1, 3584), lambda i: (0, 0)),
            pl.BlockSpec((256, 128), lambda i: (0, 0)),
            pl.BlockSpec((256, 128), lambda i: (0, 0)),
            pl.BlockSpec((_ROWS, 1), lambda i: (0, 0)),
        ],
        out_specs=[
            pl.BlockSpec((_ROWS, 1792), lambda i: (i, 0)),
            pl.BlockSpec((1, 1, 1792), lambda i: (i, 0, 0)),
            pl.BlockSpec((1, 1, 1792), lambda i: (i, 0, 0)),
        ],
        out_shape=[
            jax.ShapeDtypeStruct((B * 8, 1792), f32),
            jax.ShapeDtypeStruct((nT, 1, 1792), f32),
            jax.ShapeDtypeStruct((nT, 1, 1792), f32),
        ],
        compiler_params=cparams,
    )(xs, W0s, sh0t, W1a, W1b, mask)
    s1c = jnp.sum(s1, axis=(0, 1)).reshape(28, 64).sum(axis=0)
    q1c = jnp.sum(q1, axis=(0, 1)).reshape(28, 64).sum(axis=0)
    sc1, sh1 = _scale_shift(s1c, q1c, float(B * 14 * 14), g1, be1)
    sc1t = jnp.tile(sc1, 28).reshape(1, 1792)
    sh1t = jnp.tile(sh1, 28).reshape(1, 1792)

    h2p, s2, q2 = pl.pallas_call(
        _k_stage2,
        grid=(nT,),
        in_specs=[
            pl.BlockSpec((_ROWS, 1792), lambda i: (i, 0)),
            pl.BlockSpec((1, 1792), lambda i: (0, 0)),
            pl.BlockSpec((1, 1792), lambda i: (0, 0)),
            pl.BlockSpec((256, 128), lambda i: (0, 0)),
            pl.BlockSpec((_ROWS, 1), lambda i: (0, 0)),
        ],
        out_specs=[
            pl.BlockSpec((_ROWS, 896), lambda i: (i, 0)),
            pl.BlockSpec((1, 1, 896), lambda i: (i, 0, 0)),
            pl.BlockSpec((1, 1, 896), lambda i: (i, 0, 0)),
        ],
        out_shape=[
            jax.ShapeDtypeStruct((B * 8, 896), f32),
            jax.ShapeDtypeStruct((nT, 1, 896), f32),
            jax.ShapeDtypeStruct((nT, 1, 896), f32),
        ],
        compiler_params=cparams,
    )(h1p, sc1t, sh1t, W2g, mask)
    s2c = jnp.sum(s2, axis=(0, 1)).reshape(7, 128).sum(axis=0)
    q2c = jnp.sum(q2, axis=(0, 1)).reshape(7, 128).sum(axis=0)
    sc2, sh2 = _scale_shift(s2c, q2c, float(B * 7 * 7), g2, be2)

    out = pl.pallas_call(
        _k_stage3,
        grid=(nT,),
        in_specs=[
            pl.BlockSpec((_ROWS, 896), lambda i: (i, 0)),
            pl.BlockSpec((1, 896), lambda i: (0, 0)),
            pl.BlockSpec((1, 896), lambda i: (0, 0)),
            pl.BlockSpec((_T, _ROWS), lambda i: (0, 0)),
            pl.BlockSpec((128, 10), lambda i: (0, 0)),
            pl.BlockSpec((1, 10), lambda i: (0, 0)),
        ],
        out_specs=pl.BlockSpec((_T, 10), lambda i: (i, 0)),
        out_shape=jax.ShapeDtypeStruct((B, 10), f32),
        compiler_params=cparams,
    )(h2p, jnp.tile(sc2, 7).reshape(1, 896), jnp.tile(sh2, 7).reshape(1, 896),
      apool, wfcT, bfc.reshape(1, 10))
    return out
